# Initial kernel scaffold; baseline (speedup 1.0000x reference)
import jax
import jax.numpy as jnp
from jax.experimental import pallas as pl


def kernel(x, edge_index_cites, edge_index_cited_by, W_in, b_in, W1s_c, W1d_c, att1s_c, att1d_c, bias1_c, W1s_cb, W1d_cb, att1s_cb, att1d_cb, bias1_cb, W2_c, att2s_c, att2d_c, bias2_c, W2_cb, att2s_cb, att2d_cb, bias2_cb, A1, b1_sem, a2, Wc, bc):
    raise NotImplementedError("write your pallas kernel here")


# speedup vs baseline: 1.0073x; 1.0073x over previous
import functools

import jax
import jax.numpy as jnp
from jax.experimental import pallas as pl

_N = 10000
_D = 128
_HID = 64
_HEADS = 4
_OUT = 64


def _proj_body(x_ref, w_ref, b_ref, o_ref):
    o_ref[...] = jax.nn.relu(
        jnp.dot(x_ref[...], w_ref[...], preferred_element_type=jnp.float32)
        + b_ref[...]
    )


def _h0_pallas(x, W_in, b_in):
    blk = 1000
    return pl.pallas_call(
        _proj_body,
        grid=(_N // blk,),
        in_specs=[
            pl.BlockSpec((blk, _D), lambda i: (i, 0)),
            pl.BlockSpec((_D, _HID), lambda i: (0, 0)),
            pl.BlockSpec((1, _HID), lambda i: (0, 0)),
        ],
        out_specs=pl.BlockSpec((blk, _HID), lambda i: (i, 0)),
        out_shape=jax.ShapeDtypeStruct((_N, _HID), jnp.float32),
    )(x, W_in, b_in.reshape(1, _HID))


def _segment_softmax(alpha, dst, n):
    m = jax.ops.segment_max(alpha, dst, num_segments=n)
    m = jnp.where(jnp.isfinite(m), m, 0.0)
    e = jnp.exp(alpha - m[dst])
    s = jax.ops.segment_sum(e, dst, num_segments=n)
    return e / (s[dst] + 1e-16)


def _gat(x_src, x_dst, W_src, W_dst, att_s, att_d, bias, edge_index, n, h, c):
    xs = (x_src @ W_src).reshape(n, h, c)
    xd = (x_dst @ W_dst).reshape(n, h, c)
    a_s = (xs * att_s[None]).sum(-1)
    a_d = (xd * att_d[None]).sum(-1)
    src = edge_index[0]
    dst = edge_index[1]
    alpha = jax.nn.leaky_relu(a_s[src] + a_d[dst], 0.2)
    alpha = _segment_softmax(alpha, dst, n)
    msg = xs[src] * alpha[:, :, None]
    out = jax.ops.segment_sum(msg, dst, num_segments=n)
    return out.reshape(n, h * c) + bias


def kernel(x, edge_index_cites, edge_index_cited_by, W_in, b_in, W1s_c, W1d_c, att1s_c, att1d_c, bias1_c, W1s_cb, W1d_cb, att1s_cb, att1d_cb, bias1_cb, W2_c, att2s_c, att2d_c, bias2_c, W2_cb, att2s_cb, att2d_cb, bias2_cb, A1, b1_sem, a2, Wc, bc):
    h0 = _h0_pallas(x, W_in, b_in)
    h1 = _gat(h0, h0, W1s_c, W1d_c, att1s_c, att1d_c, bias1_c, edge_index_cites, _N, _HEADS, _HID)
    h1 = h1 + _gat(h0, h0, W1s_cb, W1d_cb, att1s_cb, att1d_cb, bias1_cb, edge_index_cited_by, _N, _HEADS, _HID)
    h1 = jax.nn.elu(h1)
    h2 = _gat(h1, h1, W2_c, W2_c, att2s_c, att2d_c, bias2_c, edge_index_cites, _N, 1, _OUT)
    h2 = h2 + _gat(h1, h1, W2_cb, W2_cb, att2s_cb, att2d_cb, bias2_cb, edge_index_cited_by, _N, 1, _OUT)
    scores = jnp.tanh(h2 @ A1 + b1_sem) @ a2
    w = jax.nn.sigmoid(scores)
    xf = w * h2
    logits = xf @ Wc + bc
    return logits


# speedup vs baseline: 35.1926x; 34.9365x over previous
import functools

import jax
import jax.numpy as jnp
from jax import lax
from jax.experimental import pallas as pl
from jax.experimental.pallas import tpu as pltpu
from jax.experimental.pallas import tpu_sc as plsc

_N = 10000
_E = 320000
_D = 128
_HID = 64
_HEADS = 4
_OUT = 64
_NC = 7

_SUB = 80
_ROWS_PER_TILE = 624


def _make_edge_agg(row, nheads, split_core_edges, tcount, chunk):
    data_cols = nheads * 64
    assert row == data_cols + 16
    nsub = chunk // _SUB
    e_per_tec = _E // (32 if split_core_edges else 16)
    iters = e_per_tec // chunk
    mesh = plsc.VectorSubcoreMesh(core_axis_name="c", subcore_axis_name="s")

    @functools.partial(
        pl.kernel,
        mesh=mesh,
        compiler_params=pltpu.CompilerParams(use_tc_tiling_on_sc=False),
        out_type=jax.ShapeDtypeStruct((2, _N, row), jnp.float32),
        scratch_types=[
            pltpu.VMEM_SHARED((_N, row), jnp.float32),
            pltpu.VMEM((chunk,), jnp.int32),
        ] + [pltpu.VMEM((_SUB,), jnp.int32) for _ in range(nsub)] + [
            pltpu.VMEM((chunk, row), jnp.float32),
            pltpu.VMEM((chunk, 16), jnp.float32),
            pltpu.SemaphoreType.DMA,
        ],
    )
    def agg(r_hbm, d_hbm, src_hbm, dst_hbm, z_hbm, out_hbm,
            acc, srcv, *rest):
        dstvs = rest[:nsub]
        rows, dvals, sem = rest[nsub:]
        c = lax.axis_index("c")
        s = lax.axis_index("s")
        pltpu.sync_copy(z_hbm.at[pl.ds(s * _ROWS_PER_TILE, _ROWS_PER_TILE)],
                        acc.at[pl.ds(s * _ROWS_PER_TILE, _ROWS_PER_TILE)])

        @pl.when(s == 15)
        def _():
            pltpu.sync_copy(z_hbm.at[pl.ds(_N - 16, 16)],
                            acc.at[pl.ds(_N - 16, 16)])

        plsc.subcore_barrier()

        if split_core_edges:
            base = (c * 16 + s) * e_per_tec
        else:
            base = s * e_per_tec
        tix = c if tcount == 2 else 0
        lanes = lax.iota(jnp.int32, 16)

        def step(i, carry):
            eo = base + i * chunk
            pltpu.sync_copy(src_hbm.at[pl.ds(eo, chunk)], srcv)
            for j in range(nsub):
                pltpu.sync_copy(dst_hbm.at[pl.ds(eo + j * _SUB, _SUB)],
                                dstvs[j])
            cps = []
            for j in range(nsub):
                cps.append(pltpu.async_copy(
                    r_hbm.at[tix].at[srcv.at[pl.ds(j * _SUB, _SUB)]],
                    rows.at[pl.ds(j * _SUB, _SUB)], sem))
                cps.append(pltpu.async_copy(
                    d_hbm.at[tix].at[dstvs[j]],
                    dvals.at[pl.ds(j * _SUB, _SUB)], sem))
            for cp in cps:
                cp.wait()

            tl = pl.ds(data_cols, 16)

            def scale(jj, carry2):
                tail = rows[jj, tl]
                a = tail + dvals[jj]
                v = jnp.where(a >= 0.0, a, 0.2 * a)
                w = jnp.exp(v)
                ws = [w[h] for h in range(nheads)]
                for h in range(nheads):
                    for q in range(4):
                        sl = pl.ds(h * 64 + q * 16, 16)
                        rows[jj, sl] = rows[jj, sl] * ws[h]
                m = jnp.zeros((16,), jnp.float32)
                for h in range(nheads):
                    m = jnp.where(lanes == nheads + h, ws[h], m)
                rows[jj, tl] = tail * m
                return carry2

            lax.fori_loop(0, chunk, scale, 0)
            for j in range(nsub):
                pltpu.sync_copy(rows.at[pl.ds(j * _SUB, _SUB)],
                                acc.at[dstvs[j]], add=True)
            return carry

        lax.fori_loop(0, iters, step, 0)
        plsc.subcore_barrier()
        pltpu.sync_copy(acc.at[pl.ds(s * _ROWS_PER_TILE, _ROWS_PER_TILE)],
                        out_hbm.at[c].at[pl.ds(s * _ROWS_PER_TILE,
                                               _ROWS_PER_TILE)])

        @pl.when(s == 15)
        def _():
            pltpu.sync_copy(acc.at[pl.ds(_N - 16, 16)],
                            out_hbm.at[c].at[pl.ds(_N - 16, 16)])

    return agg


_agg_l1 = _make_edge_agg(144, 2, split_core_edges=False, tcount=2, chunk=160)
_agg_l2 = _make_edge_agg(80, 1, split_core_edges=True, tcount=1, chunk=400)


_B = 2000
_NB = _N // _B


def _k1_body(x_ref, win_ref, bin_ref,
             wrc_ref, crc_ref, wdc_ref,
             wrcb_ref, crcb_ref, wdcb_ref,
             rc_ref, dc_ref, rcb_ref, dcb_ref):
    h0 = jax.nn.relu(
        jnp.dot(x_ref[...], win_ref[...], preferred_element_type=jnp.float32)
        + bin_ref[...])
    rc_ref[0] = jnp.dot(h0, wrc_ref[0],
                        preferred_element_type=jnp.float32) + crc_ref[0]
    dc_ref[0] = jnp.dot(h0, wdc_ref[0], preferred_element_type=jnp.float32)
    rcb_ref[0] = jnp.dot(h0, wrcb_ref[0],
                         preferred_element_type=jnp.float32) + crcb_ref[0]
    dcb_ref[0] = jnp.dot(h0, wdcb_ref[0], preferred_element_type=jnp.float32)


def _k1(x, w_in, b_in, wrc, crc, wdc, wrcb, crcb, wdcb):
    return pl.pallas_call(
        _k1_body,
        grid=(_NB, 2),
        in_specs=[
            pl.BlockSpec((_B, _D), lambda i, c: (i, 0)),
            pl.BlockSpec((_D, _HID), lambda i, c: (0, 0)),
            pl.BlockSpec((1, _HID), lambda i, c: (0, 0)),
            pl.BlockSpec((1, _HID, 144), lambda i, c: (c, 0, 0)),
            pl.BlockSpec((1, 1, 144), lambda i, c: (c, 0, 0)),
            pl.BlockSpec((1, _HID, 16), lambda i, c: (c, 0, 0)),
            pl.BlockSpec((1, _HID, 144), lambda i, c: (c, 0, 0)),
            pl.BlockSpec((1, 1, 144), lambda i, c: (c, 0, 0)),
            pl.BlockSpec((1, _HID, 16), lambda i, c: (c, 0, 0)),
        ],
        out_specs=[
            pl.BlockSpec((1, _B, 144), lambda i, c: (c, i, 0)),
            pl.BlockSpec((1, _B, 16), lambda i, c: (c, i, 0)),
            pl.BlockSpec((1, _B, 144), lambda i, c: (c, i, 0)),
            pl.BlockSpec((1, _B, 16), lambda i, c: (c, i, 0)),
        ],
        out_shape=[
            jax.ShapeDtypeStruct((2, _N, 144), jnp.float32),
            jax.ShapeDtypeStruct((2, _N, 16), jnp.float32),
            jax.ShapeDtypeStruct((2, _N, 144), jnp.float32),
            jax.ShapeDtypeStruct((2, _N, 16), jnp.float32),
        ],
    )(x, w_in, b_in, wrc, crc, wdc, wrcb, crcb, wdcb)


def _norm_l1(accv, s2):
    parts = []
    for c in (0, 1):
        num = accv[c][:, :128]
        rec = 1.0 / (accv[c][:, 128:144] + 1e-16)
        parts.append(num * jnp.dot(rec, s2,
                                   preferred_element_type=jnp.float32))
    return jnp.concatenate(parts, axis=-1)


def _k2_body(ac_ref, acb_ref, s2_ref, b1c_ref, b1cb_ref,
             wzc_ref, czc_ref, wd2c_ref, wzcb_ref, czcb_ref, wd2cb_ref,
             r2c_ref, d2c_ref, r2cb_ref, d2cb_ref):
    s2 = s2_ref[...]
    g = (_norm_l1(ac_ref[...], s2) + b1c_ref[...]
         + _norm_l1(acb_ref[...], s2) + b1cb_ref[...])
    h1 = jnp.where(g > 0.0, g, jnp.exp(jnp.minimum(g, 0.0)) - 1.0)
    r2c_ref[0] = jnp.dot(h1, wzc_ref[...],
                         preferred_element_type=jnp.float32) + czc_ref[...]
    d2c_ref[0] = jnp.dot(h1, wd2c_ref[...], preferred_element_type=jnp.float32)
    r2cb_ref[0] = jnp.dot(h1, wzcb_ref[...],
                          preferred_element_type=jnp.float32) + czcb_ref[...]
    d2cb_ref[0] = jnp.dot(h1, wd2cb_ref[...],
                          preferred_element_type=jnp.float32)


def _k2(ac, acb, s2, b1c, b1cb, wzc, czc, wd2c, wzcb, czcb, wd2cb):
    full = lambda *shape: pl.BlockSpec(shape, lambda i: (0,) * len(shape))
    return pl.pallas_call(
        _k2_body,
        grid=(_NB,),
        in_specs=[
            pl.BlockSpec((2, _B, 144), lambda i: (0, i, 0)),
            pl.BlockSpec((2, _B, 144), lambda i: (0, i, 0)),
            full(16, 128), full(1, 256), full(1, 256),
            full(256, 80), full(1, 80), full(256, 16),
            full(256, 80), full(1, 80), full(256, 16),
        ],
        out_specs=[
            pl.BlockSpec((1, _B, 80), lambda i: (0, i, 0)),
            pl.BlockSpec((1, _B, 16), lambda i: (0, i, 0)),
            pl.BlockSpec((1, _B, 80), lambda i: (0, i, 0)),
            pl.BlockSpec((1, _B, 16), lambda i: (0, i, 0)),
        ],
        out_shape=[
            jax.ShapeDtypeStruct((1, _N, 80), jnp.float32),
            jax.ShapeDtypeStruct((1, _N, 16), jnp.float32),
            jax.ShapeDtypeStruct((1, _N, 80), jnp.float32),
            jax.ShapeDtypeStruct((1, _N, 16), jnp.float32),
        ],
    )(ac, acb, s2, b1c, b1cb, wzc, czc, wd2c, wzcb, czcb, wd2cb)


def _part_l2(accv, s1, bias):
    num = accv[0][:, :64] + accv[1][:, :64]
    den = accv[0][:, 64:80] + accv[1][:, 64:80]
    rec = 1.0 / (den + 1e-16)
    return num * jnp.dot(rec, s1, preferred_element_type=jnp.float32) + bias


def _k3_body(ac_ref, acb_ref, s1_ref, b2c_ref, b2cb_ref,
             a1_ref, b1s_ref, a2_ref, wcp_ref, bcp_ref, o_ref):
    s1 = s1_ref[...]
    h2 = (_part_l2(ac_ref[...], s1, b2c_ref[...])
          + _part_l2(acb_ref[...], s1, b2cb_ref[...]))
    t = jnp.tanh(jnp.dot(h2, a1_ref[...],
                         preferred_element_type=jnp.float32) + b1s_ref[...])
    sc = jnp.sum(t * a2_ref[...], axis=-1, keepdims=True)
    w = jax.nn.sigmoid(sc)
    o_ref[...] = jnp.dot(h2 * w, wcp_ref[...],
                         preferred_element_type=jnp.float32) + bcp_ref[...]


def _k3(ac, acb, s1, b2c, b2cb, a1, b1s, a2r, wcp, bcp):
    full = lambda *shape: pl.BlockSpec(shape, lambda i: (0,) * len(shape))
    return pl.pallas_call(
        _k3_body,
        grid=(_NB,),
        in_specs=[
            pl.BlockSpec((2, _B, 80), lambda i: (0, i, 0)),
            pl.BlockSpec((2, _B, 80), lambda i: (0, i, 0)),
            full(16, 64), full(1, 64), full(1, 64),
            full(64, 128), full(1, 128), full(1, 128),
            full(64, 128), full(1, 128),
        ],
        out_specs=pl.BlockSpec((_B, 128), lambda i: (i, 0)),
        out_shape=jax.ShapeDtypeStruct((_N, 128), jnp.float32),
    )(ac, acb, s1, b2c, b2cb, a1, b1s, a2r, wcp, bcp)


def _fold_l1(w_s, w_d, att_s, att_d):
    a_src = jnp.einsum('khd,hd->kh', w_s.reshape(_HID, _HEADS, _HID), att_s)
    a_dst = jnp.einsum('khd,hd->kh', w_d.reshape(_HID, _HEADS, _HID), att_d)
    z14 = jnp.zeros((_HID, 14), jnp.float32)
    wr = jnp.stack([
        jnp.concatenate([w_s[:, :128], a_src[:, 0:2], z14], axis=1),
        jnp.concatenate([w_s[:, 128:], a_src[:, 2:4], z14], axis=1),
    ])
    cr = jnp.zeros((2, 1, 144), jnp.float32).at[:, :, 130:132].set(1.0)
    wd = jnp.stack([
        jnp.pad(a_dst[:, 0:2], ((0, 0), (0, 14))),
        jnp.pad(a_dst[:, 2:4], ((0, 0), (0, 14))),
    ])
    return wr, cr, wd


def _fold_l2(w2, att_s, att_d):
    wz = jnp.concatenate([
        w2, w2 @ att_s[0][:, None],
        jnp.zeros((_HEADS * _HID, 15), jnp.float32),
    ], axis=1)
    cz = jnp.zeros((1, 80), jnp.float32).at[0, 65].set(1.0)
    wd2 = jnp.pad(w2 @ att_d[0][:, None], ((0, 0), (0, 15)))
    return wz, cz, wd2


def kernel(x, edge_index_cites, edge_index_cited_by, W_in, b_in,
           W1s_c, W1d_c, att1s_c, att1d_c, bias1_c,
           W1s_cb, W1d_cb, att1s_cb, att1d_cb, bias1_cb,
           W2_c, att2s_c, att2d_c, bias2_c,
           W2_cb, att2s_cb, att2d_cb, bias2_cb,
           A1, b1_sem, a2, Wc, bc):
    wrc, crc, wdc = _fold_l1(W1s_c, W1d_c, att1s_c, att1d_c)
    wrcb, crcb, wdcb = _fold_l1(W1s_cb, W1d_cb, att1s_cb, att1d_cb)
    wzc, czc, wd2c = _fold_l2(W2_c, att2s_c, att2d_c)
    wzcb, czcb, wd2cb = _fold_l2(W2_cb, att2s_cb, att2d_cb)
    s2 = jnp.zeros((16, 128), jnp.float32)
    s2 = s2.at[2, :64].set(1.0).at[3, 64:128].set(1.0)
    s1 = jnp.zeros((16, 64), jnp.float32).at[1, :].set(1.0)
    wcp = jnp.pad(Wc, ((0, 0), (0, 128 - _NC)))
    bcp = jnp.pad(bc, (0, 128 - _NC)).reshape(1, 128)
    z144 = jnp.zeros((_N, 144), jnp.float32)
    z80 = jnp.zeros((_N, 80), jnp.float32)

    rc, dc, rcb, dcb = _k1(x, W_in, b_in.reshape(1, _HID),
                           wrc, crc, wdc, wrcb, crcb, wdcb)

    src_c = edge_index_cites[0]
    dst_c = edge_index_cites[1]
    src_cb = edge_index_cited_by[0]
    dst_cb = edge_index_cited_by[1]

    acc1_c = _agg_l1(rc, dc, src_c, dst_c, z144)
    acc1_cb = _agg_l1(rcb, dcb, src_cb, dst_cb, z144)

    r2c, d2c, r2cb, d2cb = _k2(acc1_c, acc1_cb, s2,
                               bias1_c.reshape(1, 256), bias1_cb.reshape(1, 256),
                               wzc, czc, wd2c, wzcb, czcb, wd2cb)

    acc2_c = _agg_l2(r2c, d2c, src_c, dst_c, z80)
    acc2_cb = _agg_l2(r2cb, d2cb, src_cb, dst_cb, z80)

    out = _k3(acc2_c, acc2_cb, s1,
              bias2_c.reshape(1, _OUT), bias2_cb.reshape(1, _OUT),
              A1, b1_sem.reshape(1, 128), a2.reshape(1, 128), wcp, bcp)
    return out[:, :_NC]
